# baseline (speedup 1.0000x reference)
import jax
import jax.numpy as jnp
from jax import lax
from jax.experimental import pallas as pl
from jax.experimental.pallas import tpu as pltpu
from jax.experimental.pallas import tpu_sc as plsc

F = 32
B = 16384
NC, NS, L = 2, 16, 16
NW = NC * NS
BPW = B // NW
CHUNK = 128
NCHUNK = BPW // CHUNK


def _body(u_hbm, ii_hbm, jj_hbm, x_hbm, y_hbm, out_hbm,
          idx_u, idx_i, idx_j, xu_v, yi_v, yj_v, out_v, sem):
    wid = lax.axis_index("s") * NC + lax.axis_index("c")

    pltpu.sync_copy(u_hbm.at[wid], idx_u)
    pltpu.sync_copy(ii_hbm.at[wid], idx_i)
    pltpu.sync_copy(jj_hbm.at[wid], idx_j)

    copies = []
    for c in range(NCHUNK):
        dst = pl.ds(c * CHUNK, CHUNK)
        copies.append(pltpu.async_copy(x_hbm.at[idx_u.at[c]], xu_v.at[dst], sem))
        copies.append(pltpu.async_copy(y_hbm.at[idx_i.at[c]], yi_v.at[dst], sem))
        copies.append(pltpu.async_copy(y_hbm.at[idx_j.at[c]], yj_v.at[dst], sem))
    for cp in copies:
        cp.wait()

    lane = lax.iota(jnp.int32, L)

    def group(g, carry):
        base = g * L
        acc = jnp.zeros((L,), jnp.float32)
        for k in range(L):
            b = base + k
            p = jnp.zeros((L,), jnp.float32)
            for h in range(F // L):
                sl = pl.ds(h * L, L)
                p = p + xu_v[b, sl] * (yi_v[b, sl] - yj_v[b, sl])
            s = jnp.sum(p)
            acc = jnp.where(lane == k, s, acc)
        out_v[pl.ds(base, L)] = acc
        return carry

    lax.fori_loop(0, BPW // L, group, 0)
    pltpu.sync_copy(out_v, out_hbm.at[pl.ds(wid * BPW, BPW)])


def kernel(user, item_i, item_j, x, y):
    mesh = plsc.VectorSubcoreMesh(core_axis_name="c", subcore_axis_name="s",
                                  num_cores=NC, num_subcores=NS)
    run = pl.kernel(
        _body,
        out_type=jax.ShapeDtypeStruct((B,), jnp.float32),
        mesh=mesh,
        compiler_params=pltpu.CompilerParams(needs_layout_passes=False,
                                             use_tc_tiling_on_sc=False),
        scratch_types=[
            pltpu.VMEM((NCHUNK, CHUNK), jnp.int32),
            pltpu.VMEM((NCHUNK, CHUNK), jnp.int32),
            pltpu.VMEM((NCHUNK, CHUNK), jnp.int32),
            pltpu.VMEM((BPW, F), jnp.float32),
            pltpu.VMEM((BPW, F), jnp.float32),
            pltpu.VMEM((BPW, F), jnp.float32),
            pltpu.VMEM((BPW,), jnp.float32),
            pltpu.SemaphoreType.DMA,
        ],
    )
    u = user.astype(jnp.int32).reshape(NW, NCHUNK, CHUNK)
    ii = item_i.astype(jnp.int32).reshape(NW, NCHUNK, CHUNK)
    jj = item_j.astype(jnp.int32).reshape(NW, NCHUNK, CHUNK)
    return run(u, ii, jj, x, y)
